# Initial kernel scaffold; baseline (speedup 1.0000x reference)
import jax
import jax.numpy as jnp
from jax.experimental import pallas as pl
from jax.experimental.pallas import tpu as pltpu
from jax.experimental.pallas import tpu_sc as plsc

_WINDOW = 256


def _gather_fn(num_indices: int, value_dim: int, dtype):
    mesh = plsc.VectorSubcoreMesh(core_axis_name="c", subcore_axis_name="s")

    @pl.kernel(
        out_type=jax.ShapeDtypeStruct((num_indices, value_dim), dtype),
        mesh=mesh,
    )
    def gather_kernel(w_hbm, i_hbm, o_hbm):
        def body(i_vmem, o_vmem):
            pltpu.sync_copy(w_hbm.at[i_vmem.at[0]], o_vmem)

        pltpu.emit_pipeline(
            body,
            grid=(num_indices // _WINDOW,),
            in_specs=[
                pl.BlockSpec((1, _WINDOW), index_map=lambda i: (0, i))
            ],
            out_specs=[
                pl.BlockSpec((_WINDOW, value_dim), index_map=lambda i: (i, 0))
            ],
            core_axis_name=("c", "s"),
            dimension_semantics=(pltpu.PARALLEL,),
        )(i_hbm, o_hbm)

    return gather_kernel


def kernel(x, W):
    batch, seq_len = x.shape
    num_indices = batch * seq_len
    value_dim = W.shape[1]
    idx = x.reshape(1, num_indices)
    out = _gather_fn(num_indices, value_dim, W.dtype)(W, idx)
    return out.reshape(batch, seq_len, value_dim)


# speedup vs baseline: 1.3072x; 1.3072x over previous
import functools

import jax
import jax.numpy as jnp
from jax import lax
from jax.experimental import pallas as pl
from jax.experimental.pallas import tpu as pltpu
from jax.experimental.pallas import tpu_sc as plsc

_NUM_CORES = 2
_NUM_SUBCORES = 16
_NW = _NUM_CORES * _NUM_SUBCORES
_CHUNK = 128


def _build(num_indices: int, value_dim: int):
    b_per_w = num_indices // _NW
    n_chunks = b_per_w // _CHUNK
    mesh = plsc.VectorSubcoreMesh(core_axis_name="c", subcore_axis_name="s")

    @functools.partial(
        pl.kernel,
        mesh=mesh,
        compiler_params=pltpu.CompilerParams(use_tc_tiling_on_sc=False),
        out_type=jax.ShapeDtypeStruct((num_indices, value_dim), jnp.float32),
        scratch_types=[
            pltpu.VMEM((b_per_w,), jnp.int32),
            pltpu.VMEM((_CHUNK, value_dim), jnp.float32),
            pltpu.SemaphoreType.DMA,
        ],
    )
    def gather_kernel(table_hbm, idx_hbm, out_hbm, idx_v, rows_v, sem):
        wid = lax.axis_index("s") * _NUM_CORES + lax.axis_index("c")
        base = wid * b_per_w
        pltpu.sync_copy(idx_hbm.at[pl.ds(base, b_per_w)], idx_v)

        @pl.loop(0, n_chunks)
        def _(c):
            off = c * _CHUNK
            pltpu.async_copy(
                table_hbm.at[idx_v.at[pl.ds(off, _CHUNK)]], rows_v, sem
            ).wait()
            pltpu.sync_copy(rows_v, out_hbm.at[pl.ds(base + off, _CHUNK)])

    return gather_kernel


def kernel(x, W):
    batch, seq_len = x.shape
    num_indices = batch * seq_len
    idx = x.reshape(num_indices)
    out = _build(num_indices, W.shape[1])(W, idx)
    return out.reshape(batch, seq_len, W.shape[1])


# speedup vs baseline: 1.4860x; 1.1368x over previous
import functools

import jax
import jax.numpy as jnp
from jax import lax
from jax.experimental import pallas as pl
from jax.experimental.pallas import tpu as pltpu
from jax.experimental.pallas import tpu_sc as plsc

_NUM_CORES = 2
_NUM_SUBCORES = 16
_NW = _NUM_CORES * _NUM_SUBCORES
_CHUNK = 512


def _build(num_indices: int, value_dim: int):
    b_per_w = num_indices // _NW
    n_chunks = b_per_w // _CHUNK
    assert n_chunks % 2 == 0 and n_chunks >= 4
    mesh = plsc.VectorSubcoreMesh(core_axis_name="c", subcore_axis_name="s")

    @functools.partial(
        pl.kernel,
        mesh=mesh,
        compiler_params=pltpu.CompilerParams(use_tc_tiling_on_sc=False),
        out_type=jax.ShapeDtypeStruct((num_indices, value_dim), jnp.float32),
        scratch_types=[
            pltpu.VMEM((b_per_w,), jnp.int32),
            pltpu.VMEM((_CHUNK, value_dim), jnp.float32),
            pltpu.VMEM((_CHUNK, value_dim), jnp.float32),
            pltpu.SemaphoreType.DMA,
            pltpu.SemaphoreType.DMA,
            pltpu.SemaphoreType.DMA,
            pltpu.SemaphoreType.DMA,
        ],
    )
    def gather_kernel(
        table_hbm, idx_hbm, out_hbm, idx_v, rows0, rows1, g0, g1, s0, s1
    ):
        wid = lax.axis_index("s") * _NUM_CORES + lax.axis_index("c")
        base = wid * b_per_w
        pltpu.sync_copy(idx_hbm.at[pl.ds(base, b_per_w)], idx_v)

        def fire_gather(c, rows, gsem):
            pltpu.async_copy(
                table_hbm.at[idx_v.at[pl.ds(c * _CHUNK, _CHUNK)]], rows, gsem
            )

        def wait_gather(rows, gsem):
            pltpu.make_async_copy(
                table_hbm.at[pl.ds(0, _CHUNK)], rows, gsem
            ).wait()

        def fire_store(c, rows, ssem):
            pltpu.async_copy(
                rows, out_hbm.at[pl.ds(base + c * _CHUNK, _CHUNK)], ssem
            )

        def wait_store(rows, ssem):
            pltpu.make_async_copy(
                rows, out_hbm.at[pl.ds(base, _CHUNK)], ssem
            ).wait()

        fire_gather(0, rows0, g0)
        fire_gather(1, rows1, g1)

        @pl.loop(0, n_chunks, step=2)
        def _(c):
            wait_gather(rows0, g0)
            fire_store(c, rows0, s0)
            wait_gather(rows1, g1)
            fire_store(c + 1, rows1, s1)

            @pl.when(c + 2 < n_chunks)
            def _():
                wait_store(rows0, s0)
                fire_gather(c + 2, rows0, g0)
                wait_store(rows1, s1)
                fire_gather(c + 3, rows1, g1)

        wait_store(rows0, s0)
        wait_store(rows1, s1)

    return gather_kernel


def kernel(x, W):
    batch, seq_len = x.shape
    num_indices = batch * seq_len
    idx = x.reshape(num_indices)
    out = _build(num_indices, W.shape[1])(W, idx)
    return out.reshape(batch, seq_len, W.shape[1])


# speedup vs baseline: 1.4955x; 1.0064x over previous
import functools

import jax
import jax.numpy as jnp
from jax import lax
from jax.experimental import pallas as pl
from jax.experimental.pallas import tpu as pltpu
from jax.experimental.pallas import tpu_sc as plsc

_NUM_CORES = 2
_NUM_SUBCORES = 16
_NW = _NUM_CORES * _NUM_SUBCORES
_CHUNK = 256
_NBUF = 4


def _build(num_indices: int, value_dim: int):
    b_per_w = num_indices // _NW
    n_chunks = b_per_w // _CHUNK
    assert n_chunks % _NBUF == 0 and n_chunks >= 2 * _NBUF
    mesh = plsc.VectorSubcoreMesh(core_axis_name="c", subcore_axis_name="s")

    row_buf = pltpu.VMEM((_CHUNK, value_dim), jnp.float32)

    @functools.partial(
        pl.kernel,
        mesh=mesh,
        compiler_params=pltpu.CompilerParams(use_tc_tiling_on_sc=False),
        out_type=jax.ShapeDtypeStruct((num_indices, value_dim), jnp.float32),
        scratch_types=(
            [pltpu.VMEM((b_per_w,), jnp.int32)]
            + [row_buf] * _NBUF
            + [pltpu.SemaphoreType.DMA] * (2 * _NBUF)
        ),
    )
    def gather_kernel(table_hbm, idx_hbm, out_hbm, idx_v, *bufs_and_sems):
        rows = bufs_and_sems[:_NBUF]
        gsems = bufs_and_sems[_NBUF : 2 * _NBUF]
        ssems = bufs_and_sems[2 * _NBUF :]

        wid = lax.axis_index("s") * _NUM_CORES + lax.axis_index("c")
        base = wid * b_per_w
        pltpu.sync_copy(idx_hbm.at[pl.ds(base, b_per_w)], idx_v)

        def fire_gather(c, b):
            pltpu.async_copy(
                table_hbm.at[idx_v.at[pl.ds(c * _CHUNK, _CHUNK)]],
                rows[b],
                gsems[b],
            )

        def wait_gather(b):
            pltpu.make_async_copy(
                table_hbm.at[pl.ds(0, _CHUNK)], rows[b], gsems[b]
            ).wait()

        def fire_store(c, b):
            pltpu.async_copy(
                rows[b], out_hbm.at[pl.ds(base + c * _CHUNK, _CHUNK)], ssems[b]
            )

        def wait_store(b):
            pltpu.make_async_copy(
                rows[b], out_hbm.at[pl.ds(base, _CHUNK)], ssems[b]
            ).wait()

        for b in range(_NBUF):
            fire_gather(b, b)

        @pl.loop(0, n_chunks, step=_NBUF)
        def _(c):
            for b in range(_NBUF):
                wait_gather(b)
                fire_store(c + b, b)

            @pl.when(c + _NBUF < n_chunks)
            def _():
                for b in range(_NBUF):
                    wait_store(b)
                    fire_gather(c + _NBUF + b, b)

        for b in range(_NBUF):
            wait_store(b)

    return gather_kernel


def kernel(x, W):
    batch, seq_len = x.shape
    num_indices = batch * seq_len
    idx = x.reshape(num_indices)
    out = _build(num_indices, W.shape[1])(W, idx)
    return out.reshape(batch, seq_len, W.shape[1])
